# Initial kernel scaffold; baseline (speedup 1.0000x reference)
import functools

import jax
import jax.numpy as jnp
from jax import lax
from jax.experimental import pallas as pl
from jax.experimental.pallas import tpu as pltpu
from jax.experimental.pallas import tpu_sc as plsc

_B, _T = 4, 8192
_V, _D = 8, 4
_N = _B * _T
_NC, _NS, _L = 2, 16, 16
_NW = _NC * _NS
_IDS_W = _N // _NW
_OUT_W = _IDS_W * _D
_VECS_W = _OUT_W // _L

_mesh = plsc.VectorSubcoreMesh(
    core_axis_name="c", subcore_axis_name="s", num_cores=_NC, num_subcores=_NS
)


@functools.partial(
    pl.kernel,
    out_type=jax.ShapeDtypeStruct((_N * _D,), jnp.float32),
    mesh=_mesh,
    scratch_types=[
        pltpu.VMEM((_IDS_W,), jnp.int32),
        pltpu.VMEM((_OUT_W,), jnp.float32),
        pltpu.VMEM((_V * _D,), jnp.float32),
    ],
)
def _embed_sc(ids_hbm, tab_hbm, out_hbm, ids_v, out_v, tab_v):
    wid = lax.axis_index("s") * _NC + lax.axis_index("c")
    base = wid * _IDS_W
    pltpu.sync_copy(ids_hbm.at[pl.ds(base, _IDS_W)], ids_v)
    pltpu.sync_copy(tab_hbm, tab_v)

    tab_v[pl.ds(0, _L)] = tab_v[pl.ds(0, _L)] + 2.0
    tab_v[pl.ds(_L, _L)] = tab_v[pl.ds(_L, _L)] + 2.0

    lanes = lax.iota(jnp.int32, _L)
    sub = lanes >> 2
    off = lanes & 3

    def body(j, carry):
        idrep = plsc.load_gather(ids_v, [j * 4 + sub])
        vals = plsc.load_gather(tab_v, [idrep * 4 + off])
        out_v[pl.ds(j * _L, _L)] = vals
        return carry

    lax.fori_loop(0, _VECS_W, body, 0)

    pltpu.sync_copy(out_v, out_hbm.at[pl.ds(base * _D, _OUT_W)])


def kernel(input_ids, embed):
    ids_flat = jnp.reshape(input_ids.astype(jnp.int32), (_N,))
    tab_flat = jnp.reshape(embed, (_V * _D,))
    out_flat = _embed_sc(ids_flat, tab_flat)
    return jnp.reshape(out_flat, (_B, _T, _D))


# speedup vs baseline: 2.2389x; 2.2389x over previous
import functools

import jax
import jax.numpy as jnp
from jax import lax
from jax.experimental import pallas as pl
from jax.experimental.pallas import tpu as pltpu
from jax.experimental.pallas import tpu_sc as plsc

_B, _T = 4, 8192
_V, _D = 8, 4
_N = _B * _T
_NC, _NS, _L = 2, 16, 16
_NW = _NC * _NS
_IDS_W = _N // _NW
_OUT_W = _IDS_W * _D
_VECS_W = _OUT_W // _L

_mesh = plsc.VectorSubcoreMesh(
    core_axis_name="c", subcore_axis_name="s", num_cores=_NC, num_subcores=_NS
)


@functools.partial(
    pl.kernel,
    out_type=jax.ShapeDtypeStruct((_N * _D,), jnp.float32),
    mesh=_mesh,
    scratch_types=[
        pltpu.VMEM((_IDS_W,), jnp.int32),
        pltpu.VMEM((_OUT_W,), jnp.float32),
        pltpu.VMEM((_V * _D,), jnp.float32),
    ],
    compiler_params=pltpu.CompilerParams(needs_layout_passes=False),
)
def _embed_sc(ids_hbm, tab_hbm, out_hbm, ids_v, out_v, tab_v):
    wid = lax.axis_index("s") * _NC + lax.axis_index("c")
    base = wid * _IDS_W
    pltpu.sync_copy(ids_hbm.at[pl.ds(base, _IDS_W)], ids_v)
    pltpu.sync_copy(tab_hbm, tab_v)

    tab_v[pl.ds(0, _L)] = tab_v[pl.ds(0, _L)] + 2.0
    tab_v[pl.ds(_L, _L)] = tab_v[pl.ds(_L, _L)] + 2.0

    lanes = lax.iota(jnp.int32, _L)
    sub = lanes >> 2
    off = lanes & 3

    def body(j, carry):
        idrep = plsc.load_gather(ids_v, [j * 4 + sub])
        vals = plsc.load_gather(tab_v, [idrep * 4 + off])
        out_v[pl.ds(j * _L, _L)] = vals
        return carry

    lax.fori_loop(0, _VECS_W, body, 0)

    pltpu.sync_copy(out_v, out_hbm.at[pl.ds(base * _D, _OUT_W)])


def kernel(input_ids, embed):
    ids_flat = jnp.reshape(input_ids.astype(jnp.int32), (_N,))
    tab_flat = jnp.reshape(embed, (_V * _D,))
    out_flat = _embed_sc(ids_flat, tab_flat)
    return jnp.reshape(out_flat, (_B, _T, _D))


# speedup vs baseline: 2.3361x; 1.0434x over previous
import functools

import jax
import jax.numpy as jnp
from jax import lax
from jax.experimental import pallas as pl
from jax.experimental.pallas import tpu as pltpu
from jax.experimental.pallas import tpu_sc as plsc

_B, _T = 4, 8192
_V, _D = 8, 4
_N = _B * _T
_NC, _NS, _L = 2, 16, 16
_NW = _NC * _NS
_IDS_W = _N // _NW
_OUT_W = _IDS_W * _D
_VECS_W = _OUT_W // _L

_mesh = plsc.VectorSubcoreMesh(
    core_axis_name="c", subcore_axis_name="s", num_cores=_NC, num_subcores=_NS
)


@functools.partial(
    pl.kernel,
    out_type=jax.ShapeDtypeStruct((_N * _D,), jnp.float32),
    mesh=_mesh,
    scratch_types=[
        pltpu.VMEM((_IDS_W,), jnp.int32),
        pltpu.VMEM((_OUT_W,), jnp.float32),
        pltpu.VMEM((_V * _D,), jnp.float32),
    ],
    compiler_params=pltpu.CompilerParams(needs_layout_passes=False),
)
def _embed_sc(ids_hbm, tab_hbm, out_hbm, ids_v, out_v, tab_v):
    wid = lax.axis_index("s") * _NC + lax.axis_index("c")
    base = wid * _IDS_W
    pltpu.sync_copy(ids_hbm.at[pl.ds(base, _IDS_W)], ids_v)
    pltpu.sync_copy(tab_hbm, tab_v)

    tab_v[pl.ds(0, _L)] = tab_v[pl.ds(0, _L)] + 2.0
    tab_v[pl.ds(_L, _L)] = tab_v[pl.ds(_L, _L)] + 2.0

    lanes = lax.iota(jnp.int32, _L)
    sub = lanes >> 2
    off = lanes & 3

    @plsc.parallel_loop(0, _VECS_W, unroll=8)
    def body(j):
        idrep = plsc.load_gather(ids_v, [j * 4 + sub])
        vals = plsc.load_gather(tab_v, [idrep * 4 + off])
        out_v[pl.ds(j * _L, _L)] = vals

    pltpu.sync_copy(out_v, out_hbm.at[pl.ds(base * _D, _OUT_W)])


def kernel(input_ids, embed):
    ids_flat = jnp.reshape(input_ids.astype(jnp.int32), (_N,))
    tab_flat = jnp.reshape(embed, (_V * _D,))
    out_flat = _embed_sc(ids_flat, tab_flat)
    return jnp.reshape(out_flat, (_B, _T, _D))


# speedup vs baseline: 2.7669x; 1.1844x over previous
import jax
import jax.numpy as jnp
from jax import lax
from jax.experimental import pallas as pl
from jax.experimental.pallas import tpu as pltpu
from jax.experimental.pallas import tpu_sc as plsc

_B, _T = 4, 8192
_V, _D = 8, 4
_N = _B * _T
_NC, _NS, _L = 2, 16, 16
_NW = _NC * _NS
_IDS_W = _N // _NW
_TOK_W = _T // _NW * _B
_OUT_W = _IDS_W * _D
_VECS_W = _OUT_W // _L
_W_PER_B = _T // _IDS_W

_mesh = plsc.VectorSubcoreMesh(
    core_axis_name="c", subcore_axis_name="s", num_cores=_NC, num_subcores=_NS
)


@pl.kernel(
    out_type=jax.ShapeDtypeStruct((_B, _T, _D), jnp.float32),
    mesh=_mesh,
    scratch_types=[
        pltpu.VMEM((_IDS_W,), jnp.int32),
        pltpu.VMEM((_IDS_W, _D), jnp.float32),
        pltpu.VMEM((_V, _D), jnp.float32),
        pltpu.VMEM((_V * _D,), jnp.float32),
    ],
    compiler_params=pltpu.CompilerParams(
        needs_layout_passes=False, use_tc_tiling_on_sc=False
    ),
)
def _embed_sc(ids_hbm, tab_hbm, out_hbm, ids_v, out_v, tab_raw, tab_v):
    wid = lax.axis_index("s") * _NC + lax.axis_index("c")
    b = wid // _W_PER_B
    t0 = (wid % _W_PER_B) * _IDS_W
    pltpu.sync_copy(ids_hbm.at[b, pl.ds(t0, _IDS_W)], ids_v)
    pltpu.sync_copy(tab_hbm, tab_raw)

    lanes = lax.iota(jnp.int32, _L)
    sub = lanes >> 2
    off = lanes & 3

    tab_v[pl.ds(0, _L)] = plsc.load_gather(tab_raw, [sub, off]) + 2.0
    tab_v[pl.ds(_L, _L)] = plsc.load_gather(tab_raw, [sub + 4, off]) + 2.0

    @plsc.parallel_loop(0, _VECS_W, unroll=8)
    def body(j):
        row = j * 4 + sub
        idrep = plsc.load_gather(ids_v, [row])
        vals = plsc.load_gather(tab_v, [idrep * 4 + off])
        plsc.store_scatter(out_v, [row, off], vals)

    pltpu.sync_copy(out_v, out_hbm.at[b, pl.ds(t0, _IDS_W), :])


def kernel(input_ids, embed):
    return _embed_sc(input_ids.astype(jnp.int32), embed)


# speedup vs baseline: 5.6763x; 2.0515x over previous
import jax
import jax.numpy as jnp
from jax import lax
from jax.experimental import pallas as pl
from jax.experimental.pallas import tpu as pltpu
from jax.experimental.pallas import tpu_sc as plsc

_B, _T = 4, 8192
_V, _D = 8, 4
_N = _B * _T
_NC, _NS, _L = 2, 16, 16
_NW = _NC * _NS
_IDS_W = _N // _NW
_W_PER_B = _T // _IDS_W
_TBLK = 128
_NBLK_W = _IDS_W // _TBLK
_NT = _T // _TBLK

_mesh = plsc.VectorSubcoreMesh(
    core_axis_name="c", subcore_axis_name="s", num_cores=_NC, num_subcores=_NS
)


@pl.kernel(
    out_type=jax.ShapeDtypeStruct((_B, _NT, _D, _TBLK), jnp.float32),
    mesh=_mesh,
    scratch_types=[
        pltpu.VMEM((_IDS_W,), jnp.int32),
        pltpu.VMEM((_NBLK_W, _D, _TBLK), jnp.float32),
        pltpu.VMEM((_V, _D), jnp.float32),
        pltpu.VMEM((_V * _D,), jnp.float32),
    ],
    compiler_params=pltpu.CompilerParams(
        needs_layout_passes=False, use_tc_tiling_on_sc=False
    ),
)
def _embed_sc(ids_hbm, tab_hbm, out_hbm, ids_v, out_v, tab_raw, tab_v):
    wid = lax.axis_index("s") * _NC + lax.axis_index("c")
    b = wid // _W_PER_B
    t0 = (wid % _W_PER_B) * _IDS_W
    pltpu.sync_copy(ids_hbm.at[b, pl.ds(t0, _IDS_W)], ids_v)
    pltpu.sync_copy(tab_hbm, tab_raw)

    lanes = lax.iota(jnp.int32, _L)
    row = lanes >> 2
    col = lanes & 3

    tab_v[pl.ds(0, _L)] = plsc.load_gather(tab_raw, [row, col]) + 2.0
    tab_v[pl.ds(_L, _L)] = plsc.load_gather(tab_raw, [row + 4, col]) + 2.0

    @plsc.parallel_loop(0, _IDS_W // _L, unroll=4)
    def body(k):
        ids4 = ids_v[pl.ds(k * _L, _L)] * 4
        blk = k >> 3
        base = (k & 7) * _L
        for d in range(_D):
            out_v[blk, d, pl.ds(base, _L)] = plsc.load_gather(tab_v, [ids4 + d])

    pltpu.sync_copy(
        out_v, out_hbm.at[b, pl.ds((t0 // _TBLK), _NBLK_W), :, :]
    )


def kernel(input_ids, embed):
    out = _embed_sc(input_ids.astype(jnp.int32), embed)
    return jnp.transpose(out, (0, 1, 3, 2)).reshape(_B, _T, _D)


# speedup vs baseline: 5.7836x; 1.0189x over previous
import jax
import jax.numpy as jnp
from jax import lax
from jax.experimental import pallas as pl
from jax.experimental.pallas import tpu as pltpu
from jax.experimental.pallas import tpu_sc as plsc

_B, _T = 4, 8192
_V, _D = 8, 4
_N = _B * _T
_NC, _NS, _L = 2, 16, 16
_NW = _NC * _NS
_IDS_W = _N // _NW
_W_PER_B = _T // _IDS_W
_TBLK = 128
_NBLK_W = _IDS_W // _TBLK
_NT = _T // _TBLK

_mesh = plsc.VectorSubcoreMesh(
    core_axis_name="c", subcore_axis_name="s", num_cores=_NC, num_subcores=_NS
)


@pl.kernel(
    out_type=jax.ShapeDtypeStruct((_B, _NT, _D, _TBLK), jnp.float32),
    mesh=_mesh,
    scratch_types=[
        pltpu.VMEM((_NBLK_W, _TBLK), jnp.int32),
        pltpu.VMEM((_NBLK_W, _D, _TBLK), jnp.float32),
        pltpu.VMEM((_D, _V), jnp.float32),
        pltpu.VMEM((_V * _D,), jnp.float32),
    ],
    compiler_params=pltpu.CompilerParams(
        needs_layout_passes=False, use_tc_tiling_on_sc=False
    ),
)
def _embed_sc(ids_hbm, tab_hbm, out_hbm, ids_v, out_v, tab_raw, tab_v):
    wid = lax.axis_index("s") * _NC + lax.axis_index("c")
    b = wid // _W_PER_B
    tb0 = (wid % _W_PER_B) * _NBLK_W
    pltpu.sync_copy(ids_hbm.at[pl.ds(tb0, _NBLK_W), b, :], ids_v)
    pltpu.sync_copy(tab_hbm, tab_raw)

    lanes = lax.iota(jnp.int32, _L)
    row = lanes >> 2
    col = lanes & 3

    tab_v[pl.ds(0, _L)] = plsc.load_gather(tab_raw, [col, row]) + 2.0
    tab_v[pl.ds(_L, _L)] = plsc.load_gather(tab_raw, [col, row + 4]) + 2.0

    @plsc.parallel_loop(0, _IDS_W // _L, unroll=4)
    def body(k):
        blk = k >> 3
        base = (k & 7) * _L
        ids4 = ids_v[blk, pl.ds(base, _L)] * 4
        for d in range(_D):
            out_v[blk, d, pl.ds(base, _L)] = plsc.load_gather(tab_v, [ids4 + d])

    pltpu.sync_copy(out_v, out_hbm.at[b, pl.ds(tb0, _NBLK_W), :, :])


def kernel(input_ids, embed):
    ids3 = jnp.transpose(
        input_ids.astype(jnp.int32).reshape(_B, _NT, _TBLK), (1, 0, 2)
    )
    out = _embed_sc(ids3, embed.T)
    return jnp.transpose(out, (0, 1, 3, 2)).reshape(_B, _T, _D)


# speedup vs baseline: 5.8132x; 1.0051x over previous
import jax
import jax.numpy as jnp
from jax import lax
from jax.experimental import pallas as pl
from jax.experimental.pallas import tpu as pltpu
from jax.experimental.pallas import tpu_sc as plsc

_B, _T = 4, 8192
_V, _D = 8, 4
_N = _B * _T
_NC, _NS, _L = 2, 16, 16
_NW = _NC * _NS
_IDS_W = _N // _NW
_W_PER_B = _T // _IDS_W
_TBLK = 128
_NBLK_W = _IDS_W // _TBLK
_NT = _T // _TBLK

_mesh = plsc.VectorSubcoreMesh(
    core_axis_name="c", subcore_axis_name="s", num_cores=_NC, num_subcores=_NS
)


@pl.kernel(
    out_type=jax.ShapeDtypeStruct((_B, _NT, _D, _TBLK), jnp.float32),
    mesh=_mesh,
    scratch_types=[
        pltpu.VMEM((_NBLK_W, _TBLK), jnp.int32),
        pltpu.VMEM((_NBLK_W, _D, _TBLK), jnp.float32),
        pltpu.VMEM((_D, _V), jnp.float32),
        pltpu.VMEM((_V * _D,), jnp.float32),
    ],
    compiler_params=pltpu.CompilerParams(
        needs_layout_passes=False, use_tc_tiling_on_sc=False
    ),
)
def _embed_sc(ids_hbm, tab_hbm, out_hbm, ids_v, out_v, tab_raw, tab_v):
    wid = lax.axis_index("s") * _NC + lax.axis_index("c")
    b = wid // _W_PER_B
    tb0 = (wid % _W_PER_B) * _NBLK_W
    pltpu.sync_copy(ids_hbm.at[pl.ds(tb0, _NBLK_W), b, :], ids_v)
    pltpu.sync_copy(tab_hbm, tab_raw)

    lanes = lax.iota(jnp.int32, _L)
    row = lanes >> 2
    col = lanes & 3

    tab_v[pl.ds(0, _L)] = plsc.load_gather(tab_raw, [col, row]) + 2.0
    tab_v[pl.ds(_L, _L)] = plsc.load_gather(tab_raw, [col, row + 4]) + 2.0

    @plsc.parallel_loop(0, _IDS_W // _L, unroll=2)
    def body(k):
        blk = k >> 3
        base = (k & 7) * _L
        ids4 = ids_v[blk, pl.ds(base, _L)] * 4
        for d in range(_D):
            out_v[blk, d, pl.ds(base, _L)] = plsc.load_gather(tab_v, [ids4 + d])

    pltpu.sync_copy(out_v, out_hbm.at[b, pl.ds(tb0, _NBLK_W), :, :])


def kernel(input_ids, embed):
    ids3 = jnp.transpose(
        input_ids.astype(jnp.int32).reshape(_B, _NT, _TBLK), (1, 0, 2)
    )
    out = _embed_sc(ids3, embed.T)
    return jnp.transpose(out, (0, 1, 3, 2)).reshape(_B, _T, _D)
